# baseline (speedup 1.0000x reference)
import functools

import jax
import jax.numpy as jnp
from jax import lax
from jax.experimental import pallas as pl
from jax.experimental.pallas import tpu as pltpu
from jax.experimental.pallas import tpu_sc as plsc

VOCAB = 1000000
EMB = 64
BATCH = 16384
HIST = 200
SCALE = EMB ** (-0.5)

NC = 2
NS = 16
NW = NC * NS

B = BATCH * HIST
B_PER_W = B // NW
CHUNK = 512
IDX_W = 128
GATHERS = CHUNK // IDX_W
NCHUNK = B_PER_W // CHUNK
IDX_ROWS_PER_W = B_PER_W // IDX_W


def _sc_body(x_hbm, table_hbm, out_hbm, idx_v, rows_v, gsem):
    c = lax.axis_index("c")
    s = lax.axis_index("s")
    wid = s * NC + c
    base = wid * B_PER_W
    idx_row_base = wid * IDX_ROWS_PER_W

    @pl.loop(0, NCHUNK)
    def _chunk(g):
        pltpu.sync_copy(x_hbm.at[pl.ds(idx_row_base + g * GATHERS, GATHERS)],
                        idx_v)
        descs = []
        for j in range(GATHERS):
            descs.append(pltpu.async_copy(
                table_hbm.at[idx_v.at[j]],
                rows_v.at[pl.ds(j * IDX_W, IDX_W)],
                gsem))
        for d in descs:
            d.wait()

        @pl.loop(0, CHUNK, unroll=8)
        def _scale(r):
            for jj in range(EMB // 16):
                sl = pl.ds(jj * 16, 16)
                rows_v[r, sl] = rows_v[r, sl] * SCALE

        pltpu.sync_copy(rows_v, out_hbm.at[pl.ds(base + g * CHUNK, CHUNK)])


@jax.jit
def _run(x2, table):
    mesh = plsc.VectorSubcoreMesh(core_axis_name="c", subcore_axis_name="s",
                                  num_cores=NC, num_subcores=NS)
    f = pl.kernel(
        _sc_body,
        out_type=jax.ShapeDtypeStruct((B, EMB), jnp.float32),
        mesh=mesh,
        compiler_params=pltpu.CompilerParams(use_tc_tiling_on_sc=False),
        scratch_types=[
            pltpu.VMEM((GATHERS, IDX_W), jnp.int32),
            pltpu.VMEM((CHUNK, EMB), jnp.float32),
            pltpu.SemaphoreType.DMA,
        ],
    )
    return f(x2, table)


def kernel(x, table):
    x2 = x.astype(jnp.int32).reshape(B // IDX_W, IDX_W)
    out = _run(x2, table)
    return out.reshape(BATCH, HIST, EMB)


# speedup vs baseline: 1.1387x; 1.1387x over previous
import functools

import jax
import jax.numpy as jnp
from jax import lax
from jax.experimental import pallas as pl
from jax.experimental.pallas import tpu as pltpu
from jax.experimental.pallas import tpu_sc as plsc

VOCAB = 1000000
EMB = 64
BATCH = 16384
HIST = 200
SCALE = EMB ** (-0.5)

NC = 2
NS = 16
NW = NC * NS

B = BATCH * HIST
B_PER_W = B // NW
CHUNK = 256
IDX_W = 128
GATHERS = CHUNK // IDX_W
NCHUNK = B_PER_W // CHUNK
IDX_ROWS_PER_W = B_PER_W // IDX_W
NBUF = 4


def _sc_body(x_hbm, table_hbm, out_hbm, idx_v, rows_v,
             gs0, gs1, gs2, gs3, ss0, ss1, ss2, ss3):
    gs = [gs0, gs1, gs2, gs3]
    ss = [ss0, ss1, ss2, ss3]
    c = lax.axis_index("c")
    s = lax.axis_index("s")
    wid = s * NC + c
    base = wid * B_PER_W
    idx_row_base = wid * IDX_ROWS_PER_W

    def stage_and_fire_gather(g, b):
        pltpu.sync_copy(
            x_hbm.at[pl.ds(idx_row_base + g * GATHERS, GATHERS)],
            idx_v.at[pl.ds(b * GATHERS, GATHERS)])
        for j in range(GATHERS):
            pltpu.async_copy(
                table_hbm.at[idx_v.at[b * GATHERS + j]],
                rows_v.at[pl.ds(b * CHUNK + j * IDX_W, IDX_W)],
                gs[b])

    def wait_gather(b):
        for j in range(GATHERS):
            pltpu.make_async_copy(
                table_hbm.at[idx_v.at[b * GATHERS + j]],
                rows_v.at[pl.ds(b * CHUNK + j * IDX_W, IDX_W)],
                gs[b]).wait()

    def fire_scatter(g, b):
        pltpu.async_copy(rows_v.at[pl.ds(b * CHUNK, CHUNK)],
                         out_hbm.at[pl.ds(base + g * CHUNK, CHUNK)],
                         ss[b])

    def wait_scatter(g, b):
        pltpu.make_async_copy(rows_v.at[pl.ds(b * CHUNK, CHUNK)],
                              out_hbm.at[pl.ds(base + g * CHUNK, CHUNK)],
                              ss[b]).wait()

    stage_and_fire_gather(0, 0)
    stage_and_fire_gather(1, 1)

    @pl.loop(0, NCHUNK // NBUF)
    def _outer(go):
        for b in range(NBUF):
            g = go * NBUF + b
            b2 = (b + 2) % NBUF

            @pl.when(g >= 2)
            def _():
                wait_scatter(g - 2, b2)

            @pl.when(g + 2 < NCHUNK)
            def _():
                stage_and_fire_gather(g + 2, b2)

            wait_gather(b)

            @pl.loop(0, CHUNK, unroll=8)
            def _scale(r):
                for jj in range(EMB // 16):
                    sl = pl.ds(jj * 16, 16)
                    rows_v[b * CHUNK + r, sl] = rows_v[b * CHUNK + r, sl] * SCALE

            fire_scatter(g, b)

    wait_scatter(NCHUNK - 2, (NCHUNK - 2) % NBUF)
    wait_scatter(NCHUNK - 1, (NCHUNK - 1) % NBUF)


@jax.jit
def _run(x2, table):
    mesh = plsc.VectorSubcoreMesh(core_axis_name="c", subcore_axis_name="s",
                                  num_cores=NC, num_subcores=NS)
    f = pl.kernel(
        _sc_body,
        out_type=jax.ShapeDtypeStruct((B, EMB), jnp.float32),
        mesh=mesh,
        compiler_params=pltpu.CompilerParams(use_tc_tiling_on_sc=False),
        scratch_types=[
            pltpu.VMEM((NBUF * GATHERS, IDX_W), jnp.int32),
            pltpu.VMEM((NBUF * CHUNK, EMB), jnp.float32),
        ] + [pltpu.SemaphoreType.DMA] * (2 * NBUF),
    )
    return f(x2, table)


def kernel(x, table):
    x2 = x.astype(jnp.int32).reshape(B // IDX_W, IDX_W)
    out = _run(x2, table)
    return out.reshape(BATCH, HIST, EMB)


# speedup vs baseline: 1.1473x; 1.0075x over previous
import jax
import jax.numpy as jnp
from jax import lax
from jax.experimental import pallas as pl
from jax.experimental.pallas import tpu as pltpu
from jax.experimental.pallas import tpu_sc as plsc

VOCAB = 1000000
EMB = 64
BATCH = 16384
HIST = 200
SCALE = EMB ** (-0.5)

NC = 2
NS = 16
NW = NC * NS

B = BATCH * HIST
ROWS_PER_W = BATCH // NW
XR = 2
CLOOK = XR * HIST
NCHUNK = ROWS_PER_W // XR
NBUF = 4
G_SPLIT = (128, HIST - 128)


def _pieces(b):
    out = []
    for r in range(XR):
        off = b * CLOOK + r * HIST
        out.append((off, G_SPLIT[0]))
        out.append((off + G_SPLIT[0], G_SPLIT[1]))
    return out


def _sc_body(x_hbm, table_hbm, out_hbm, idx_v, rows_v,
             gs0, gs1, gs2, gs3, ss0, ss1, ss2, ss3):
    gs = [gs0, gs1, gs2, gs3]
    ss = [ss0, ss1, ss2, ss3]
    c = lax.axis_index("c")
    s = lax.axis_index("s")
    wid = s * NC + c
    row_base = wid * ROWS_PER_W

    def stage_and_fire_gather(g, b):
        pltpu.sync_copy(
            x_hbm.at[pl.ds((row_base + g * XR) * HIST, CLOOK)],
            idx_v.at[pl.ds(b * CLOOK, CLOOK)])
        for off, ln in _pieces(b):
            pltpu.async_copy(table_hbm.at[idx_v.at[pl.ds(off, ln)]],
                             rows_v.at[pl.ds(off, ln)], gs[b])

    def wait_gather(b):
        for off, ln in _pieces(b):
            pltpu.make_async_copy(table_hbm.at[idx_v.at[pl.ds(off, ln)]],
                                  rows_v.at[pl.ds(off, ln)], gs[b]).wait()

    def fire_scatter(g, b):
        for r in range(XR):
            pltpu.async_copy(
                rows_v.at[pl.ds(b * CLOOK + r * HIST, HIST)],
                out_hbm.at[row_base + g * XR + r],
                ss[b])

    def wait_scatter(g, b):
        for r in range(XR):
            pltpu.make_async_copy(
                rows_v.at[pl.ds(b * CLOOK + r * HIST, HIST)],
                out_hbm.at[row_base + g * XR + r],
                ss[b]).wait()

    stage_and_fire_gather(0, 0)
    stage_and_fire_gather(1, 1)

    @pl.loop(0, NCHUNK // NBUF)
    def _outer(go):
        for b in range(NBUF):
            g = go * NBUF + b
            b2 = (b + 2) % NBUF

            @pl.when(g >= 2)
            def _():
                wait_scatter(g - 2, b2)

            @pl.when(g + 2 < NCHUNK)
            def _():
                stage_and_fire_gather(g + 2, b2)

            wait_gather(b)

            @pl.loop(0, CLOOK, unroll=8)
            def _scale(r):
                for jj in range(EMB // 16):
                    sl = pl.ds(jj * 16, 16)
                    rows_v[b * CLOOK + r, sl] = rows_v[b * CLOOK + r, sl] * SCALE

            fire_scatter(g, b)

    wait_scatter(NCHUNK - 2, (NCHUNK - 2) % NBUF)
    wait_scatter(NCHUNK - 1, (NCHUNK - 1) % NBUF)


@jax.jit
def _run(x_flat, table):
    mesh = plsc.VectorSubcoreMesh(core_axis_name="c", subcore_axis_name="s",
                                  num_cores=NC, num_subcores=NS)
    f = pl.kernel(
        _sc_body,
        out_type=jax.ShapeDtypeStruct((BATCH, HIST, EMB), jnp.float32),
        mesh=mesh,
        compiler_params=pltpu.CompilerParams(use_tc_tiling_on_sc=False),
        scratch_types=[
            pltpu.VMEM((NBUF * CLOOK,), jnp.int32),
            pltpu.VMEM((NBUF * CLOOK, EMB), jnp.float32),
        ] + [pltpu.SemaphoreType.DMA] * (2 * NBUF),
    )
    return f(x_flat, table)


def kernel(x, table):
    x_flat = x.astype(jnp.int32).reshape(B)
    return _run(x_flat, table)


# speedup vs baseline: 1.8515x; 1.6138x over previous
import jax
import jax.numpy as jnp
from jax import lax
from jax.experimental import pallas as pl
from jax.experimental.pallas import tpu as pltpu
from jax.experimental.pallas import tpu_sc as plsc

VOCAB = 1000000
EMB = 64
EMBP = 128
BATCH = 16384
HIST = 200
SCALE = EMB ** (-0.5)

NC = 2
NS = 16
NW = NC * NS

B = BATCH * HIST
ROWS_PER_W = BATCH // NW
CLOOK = HIST
NCHUNK = ROWS_PER_W
NBUF = 4
G_SPLIT = (128, HIST - 128)


def _pieces(b):
    off = b * CLOOK
    return [(off, G_SPLIT[0]), (off + G_SPLIT[0], G_SPLIT[1])]


def _sc_body(x_hbm, table_hbm, out_hbm, idx_v, rows_v,
             gs0, gs1, gs2, gs3, ss0, ss1, ss2, ss3):
    gs = [gs0, gs1, gs2, gs3]
    ss = [ss0, ss1, ss2, ss3]
    c = lax.axis_index("c")
    s = lax.axis_index("s")
    wid = s * NC + c
    row_base = wid * ROWS_PER_W

    def stage_and_fire_gather(g, b):
        pltpu.sync_copy(
            x_hbm.at[pl.ds((row_base + g) * HIST, CLOOK)],
            idx_v.at[pl.ds(b * CLOOK, CLOOK)])
        for off, ln in _pieces(b):
            pltpu.async_copy(
                table_hbm.at[idx_v.at[pl.ds(off, ln)]],
                rows_v.at[pl.ds(off, ln)],
                gs[b])

    def wait_gather(b):
        for off, ln in _pieces(b):
            pltpu.make_async_copy(
                table_hbm.at[idx_v.at[pl.ds(off, ln)]],
                rows_v.at[pl.ds(off, ln)],
                gs[b]).wait()

    def fire_scatter(g, b):
        pltpu.async_copy(rows_v.at[pl.ds(b * CLOOK, CLOOK)],
                         out_hbm.at[row_base + g, :, pl.ds(0, EMB)], ss[b])

    def wait_scatter(g, b):
        pltpu.make_async_copy(rows_v.at[pl.ds(b * CLOOK, CLOOK)],
                              out_hbm.at[row_base + g, :, pl.ds(0, EMB)],
                              ss[b]).wait()

    stage_and_fire_gather(0, 0)
    stage_and_fire_gather(1, 1)

    @pl.loop(0, NCHUNK // NBUF)
    def _outer(go):
        for b in range(NBUF):
            g = go * NBUF + b
            b2 = (b + 2) % NBUF

            @pl.when(g >= 2)
            def _():
                wait_scatter(g - 2, b2)

            @pl.when(g + 2 < NCHUNK)
            def _():
                stage_and_fire_gather(g + 2, b2)

            wait_gather(b)

            @pl.loop(0, CLOOK, unroll=8)
            def _scale(r):
                for jj in range(EMB // 16):
                    sl = pl.ds(jj * 16, 16)
                    rows_v[b * CLOOK + r, sl] = rows_v[b * CLOOK + r, sl] * SCALE

            fire_scatter(g, b)

    wait_scatter(NCHUNK - 2, (NCHUNK - 2) % NBUF)
    wait_scatter(NCHUNK - 1, (NCHUNK - 1) % NBUF)


@jax.jit
def _run(x_flat, table):
    mesh = plsc.VectorSubcoreMesh(core_axis_name="c", subcore_axis_name="s",
                                  num_cores=NC, num_subcores=NS)
    f = pl.kernel(
        _sc_body,
        out_type=jax.ShapeDtypeStruct((BATCH, HIST, EMBP), jnp.float32),
        mesh=mesh,
        compiler_params=pltpu.CompilerParams(use_tc_tiling_on_sc=False),
        scratch_types=[
            pltpu.VMEM((NBUF * CLOOK,), jnp.int32),
            pltpu.VMEM((NBUF * CLOOK, EMB), jnp.float32),
        ] + [pltpu.SemaphoreType.DMA] * (2 * NBUF),
    )
    return f(x_flat, table)


def kernel(x, table):
    x_flat = x.astype(jnp.int32).reshape(B)
    out = _run(x_flat, table)
    return out[:, :, :EMB]
